# baseline (speedup 1.0000x reference)
import functools

import jax
import jax.numpy as jnp
from jax import lax
from jax.experimental import pallas as pl
from jax.experimental.pallas import tpu as pltpu
from jax.experimental.pallas import tpu_sc as plsc

N = 10000
E = 320000
D = 128

NC = 2
NS = 16
NW = NC * NS
EPW = E // NW
CHUNK = 100
NCHUNK = EPW // CHUNK
NB = 3
NR = 2 * NB
NP = 10240
ROWS_PER_TILE = NP // NS
ZCOPY = 80


def _sc_partials(feature, src2, dst2):
    mesh = plsc.VectorSubcoreMesh(core_axis_name="c", subcore_axis_name="s")

    @functools.partial(
        pl.kernel,
        mesh=mesh,
        out_type=jax.ShapeDtypeStruct((NC, NP, D), jnp.float32),
        scratch_types=[
            pltpu.VMEM((NR, CHUNK), jnp.int32),
            pltpu.VMEM((NR, CHUNK), jnp.int32),
            pltpu.VMEM((NB, CHUNK, D), jnp.float32),
            pltpu.VMEM_SHARED((NP, D), jnp.float32),
            [pltpu.SemaphoreType.DMA] * NR,
            [pltpu.SemaphoreType.DMA] * NB,
            [pltpu.SemaphoreType.DMA] * NB,
        ],
    )
    def k(feat_hbm, src_hbm, dst_hbm, out_hbm, sring, dring, rows, acc,
          isems, gsems, ssems):
        cid = lax.axis_index("c")
        sid = lax.axis_index("s")
        wid = sid * NC + cid

        def zrow(r, carry):
            def zcol(j, carry2):
                rows[0, r, pl.ds(j * 16, 16)] = jnp.zeros((16,), jnp.float32)
                return carry2
            return lax.fori_loop(0, D // 16, zcol, carry)
        lax.fori_loop(0, ZCOPY, zrow, 0)
        for kk in range(ROWS_PER_TILE // ZCOPY):
            pltpu.async_copy(rows.at[0, pl.ds(0, ZCOPY)],
                             acc.at[pl.ds(sid * ROWS_PER_TILE + kk * ZCOPY, ZCOPY)],
                             gsems[0])
        for kk in range(ROWS_PER_TILE // ZCOPY):
            pltpu.make_async_copy(
                rows.at[0, pl.ds(0, ZCOPY)],
                acc.at[pl.ds(sid * ROWS_PER_TILE + kk * ZCOPY, ZCOPY)],
                gsems[0]).wait()
        plsc.subcore_barrier()

        def load_idx(i, s):
            pltpu.async_copy(src_hbm.at[wid, i], sring.at[s], isems[s])
            pltpu.async_copy(dst_hbm.at[wid, i], dring.at[s], isems[s])

        def wait_idx(i, s):
            pltpu.make_async_copy(src_hbm.at[wid, i], sring.at[s], isems[s]).wait()
            pltpu.make_async_copy(dst_hbm.at[wid, i], dring.at[s], isems[s]).wait()

        def issue_g(b, s):
            pltpu.async_copy(feat_hbm.at[sring.at[s]], rows.at[b], gsems[b])

        def wait_g(b, s):
            pltpu.make_async_copy(feat_hbm.at[sring.at[s]], rows.at[b], gsems[b]).wait()

        def scatter_a(b, s):
            pltpu.async_copy(rows.at[b], acc.at[dring.at[s]], ssems[b], add=True)

        def wait_s(b, s):
            pltpu.make_async_copy(rows.at[b], acc.at[dring.at[s]], ssems[b]).wait()

        def step(i):
            b = i % NB
            s = i % NR
            wait_g(b, s)
            scatter_a(b, s)
            if i >= 1:
                wait_s((i - 1) % NB, (i - 1) % NR)
            if i + NR - 1 < NCHUNK:
                load_idx(i + NR - 1, (i + NR - 1) % NR)
            if i + 2 < NCHUNK:
                wait_idx(i + 2, (i + 2) % NR)
                issue_g((i + 2) % NB, (i + 2) % NR)

        for s in range(NR - 1):
            load_idx(s, s)
        for b in range(2):
            wait_idx(b, b)
            issue_g(b, b)

        for i in range(NR):
            step(i)

        NGRP = (NCHUNK - 2 * NR) // NR

        def body(j, carry):
            a = NR * (j + 1)
            for u in range(NR):
                i = a + u
                b = u % NB
                s = u
                wait_g(b, s)
                scatter_a(b, s)
                wait_s((u - 1) % NB, (u - 1) % NR)
                load_idx(i + NR - 1, (u + NR - 1) % NR)
                wait_idx(i + 2, (u + 2) % NR)
                issue_g((u + 2) % NB, (u + 2) % NR)
            return carry
        lax.fori_loop(0, NGRP, body, 0)

        for i in range((NGRP + 1) * NR, NCHUNK):
            step(i)
        wait_s((NCHUNK - 1) % NB, (NCHUNK - 1) % NR)
        plsc.subcore_barrier()

        for kk in range(ROWS_PER_TILE // ZCOPY):
            r0 = sid * ROWS_PER_TILE + kk * ZCOPY
            pltpu.async_copy(acc.at[pl.ds(r0, ZCOPY)],
                             out_hbm.at[cid, pl.ds(r0, ZCOPY)], gsems[0])
        for kk in range(ROWS_PER_TILE // ZCOPY):
            r0 = sid * ROWS_PER_TILE + kk * ZCOPY
            pltpu.make_async_copy(acc.at[pl.ds(r0, ZCOPY)],
                                  out_hbm.at[cid, pl.ds(r0, ZCOPY)], gsems[0]).wait()

    return k(feature, src2, dst2)


BLK = 1000


def _tc_linear_kernel(p_ref, w_ref, b_ref, out_ref):
    x = p_ref[0] + p_ref[1]
    y = lax.dot_general(
        x, w_ref[...], (((1,), (1,)), ((), ())),
        preferred_element_type=jnp.float32,
        precision=lax.Precision.HIGHEST,
    )
    out_ref[...] = y + b_ref[...]


def _tc_linear(partials, W, b):
    return pl.pallas_call(
        _tc_linear_kernel,
        grid=(N // BLK,),
        in_specs=[
            pl.BlockSpec((NC, BLK, D), lambda i: (0, i, 0)),
            pl.BlockSpec((D, D), lambda i: (0, 0)),
            pl.BlockSpec((1, D), lambda i: (0, 0)),
        ],
        out_specs=pl.BlockSpec((BLK, D), lambda i: (i, 0)),
        out_shape=jax.ShapeDtypeStruct((N, D), jnp.float32),
    )(partials, W, b.reshape(1, D))


@jax.jit
def kernel(feature, edge_index, W, b):
    src2 = edge_index[0].reshape(NW, NCHUNK, CHUNK)
    dst2 = edge_index[1].reshape(NW, NCHUNK, CHUNK)
    partials = _sc_partials(feature, src2, dst2)
    return _tc_linear(partials, W, b)


# speedup vs baseline: 1.0421x; 1.0421x over previous
import functools

import jax
import jax.numpy as jnp
from jax import lax
from jax.experimental import pallas as pl
from jax.experimental.pallas import tpu as pltpu
from jax.experimental.pallas import tpu_sc as plsc

N = 10000
E = 320000
D = 128

NC = 2
NS = 16
NW = NC * NS
EPW = E // NW
CHUNK = 100
NCHUNK = EPW // CHUNK
NB = 3
NR = 2 * NB
NP = 10240
ROWS_PER_TILE = NP // NS
ZCOPY = 80


def _sc_partials(feature, src2, dst2):
    mesh = plsc.VectorSubcoreMesh(core_axis_name="c", subcore_axis_name="s")

    @functools.partial(
        pl.kernel,
        mesh=mesh,
        out_type=jax.ShapeDtypeStruct((NC, NP, D), jnp.float32),
        scratch_types=[
            pltpu.VMEM((NR, CHUNK), jnp.int32),
            pltpu.VMEM((NR, CHUNK), jnp.int32),
            pltpu.VMEM((NB, CHUNK, D), jnp.float32),
            pltpu.VMEM_SHARED((NP, D), jnp.float32),
            [pltpu.SemaphoreType.DMA] * NR,
            [pltpu.SemaphoreType.DMA] * NB,
        ],
    )
    def k(feat_hbm, src_hbm, dst_hbm, out_hbm, sring, dring, rows, acc,
          isems, gsems):
        cid = lax.axis_index("c")
        sid = lax.axis_index("s")
        wid = sid * NC + cid

        def zrow(r, carry):
            def zcol(j, carry2):
                rows[0, r, pl.ds(j * 16, 16)] = jnp.zeros((16,), jnp.float32)
                return carry2
            return lax.fori_loop(0, D // 16, zcol, carry)
        lax.fori_loop(0, ZCOPY, zrow, 0)
        for kk in range(ROWS_PER_TILE // ZCOPY):
            pltpu.async_copy(rows.at[0, pl.ds(0, ZCOPY)],
                             acc.at[pl.ds(sid * ROWS_PER_TILE + kk * ZCOPY, ZCOPY)],
                             gsems[0])
        for kk in range(ROWS_PER_TILE // ZCOPY):
            pltpu.make_async_copy(
                rows.at[0, pl.ds(0, ZCOPY)],
                acc.at[pl.ds(sid * ROWS_PER_TILE + kk * ZCOPY, ZCOPY)],
                gsems[0]).wait()
        plsc.subcore_barrier()

        def load_idx(i, s):
            pltpu.async_copy(src_hbm.at[wid, i], sring.at[s], isems[s])
            pltpu.async_copy(dst_hbm.at[wid, i], dring.at[s], isems[s])

        def wait_idx(i, s):
            pltpu.make_async_copy(src_hbm.at[wid, i], sring.at[s], isems[s]).wait()
            pltpu.make_async_copy(dst_hbm.at[wid, i], dring.at[s], isems[s]).wait()

        def issue_g(b, s):
            pltpu.async_copy(feat_hbm.at[sring.at[s]], rows.at[b], gsems[b])

        def wait_g(b, s):
            pltpu.make_async_copy(feat_hbm.at[sring.at[s]], rows.at[b], gsems[b]).wait()

        def scatter(b, s):
            pltpu.sync_copy(rows.at[b], acc.at[dring.at[s]], add=True)

        for s in range(NR):
            load_idx(s, s)
        for b in range(NB):
            wait_idx(b, b)
            issue_g(b, b)

        NGRP = (NCHUNK - NR) // NR

        def body(j, carry):
            a = NR * j
            for u in range(NR):
                b = u % NB
                wait_g(b, u)
                scatter(b, u)
                load_idx(a + u + NR, u)
                wait_idx(a + u + NB, (u + NB) % NR)
                issue_g(b, (u + NB) % NR)
            return carry
        lax.fori_loop(0, NGRP, body, 0)

        for i in range(NGRP * NR, NCHUNK):
            b = i % NB
            u = i % NR
            wait_g(b, u)
            scatter(b, u)
            if i + NR < NCHUNK:
                load_idx(i + NR, u)
            if i + NB < NCHUNK:
                wait_idx(i + NB, (i + NB) % NR)
                issue_g(b, (i + NB) % NR)
        plsc.subcore_barrier()

        for kk in range(ROWS_PER_TILE // ZCOPY):
            r0 = sid * ROWS_PER_TILE + kk * ZCOPY
            pltpu.async_copy(acc.at[pl.ds(r0, ZCOPY)],
                             out_hbm.at[cid, pl.ds(r0, ZCOPY)], gsems[0])
        for kk in range(ROWS_PER_TILE // ZCOPY):
            r0 = sid * ROWS_PER_TILE + kk * ZCOPY
            pltpu.make_async_copy(acc.at[pl.ds(r0, ZCOPY)],
                                  out_hbm.at[cid, pl.ds(r0, ZCOPY)], gsems[0]).wait()

    return k(feature, src2, dst2)


BLK = 1000


def _tc_linear_kernel(p_ref, w_ref, b_ref, out_ref):
    x = p_ref[0] + p_ref[1]
    y = lax.dot_general(
        x, w_ref[...], (((1,), (1,)), ((), ())),
        preferred_element_type=jnp.float32,
        precision=lax.Precision.HIGHEST,
    )
    out_ref[...] = y + b_ref[...]


def _tc_linear(partials, W, b):
    return pl.pallas_call(
        _tc_linear_kernel,
        grid=(N // BLK,),
        in_specs=[
            pl.BlockSpec((NC, BLK, D), lambda i: (0, i, 0)),
            pl.BlockSpec((D, D), lambda i: (0, 0)),
            pl.BlockSpec((1, D), lambda i: (0, 0)),
        ],
        out_specs=pl.BlockSpec((BLK, D), lambda i: (i, 0)),
        out_shape=jax.ShapeDtypeStruct((N, D), jnp.float32),
    )(partials, W, b.reshape(1, D))


@jax.jit
def kernel(feature, edge_index, W, b):
    src2 = edge_index[0].reshape(NW, NCHUNK, CHUNK)
    dst2 = edge_index[1].reshape(NW, NCHUNK, CHUNK)
    partials = _sc_partials(feature, src2, dst2)
    return _tc_linear(partials, W, b)


# speedup vs baseline: 1.0856x; 1.0418x over previous
import functools

import jax
import jax.numpy as jnp
from jax import lax
from jax.experimental import pallas as pl
from jax.experimental.pallas import tpu as pltpu
from jax.experimental.pallas import tpu_sc as plsc

N = 10000
E = 320000
D = 128

NC = 2
NS = 16
NW = NC * NS
EPW = E // NW
CHUNK = 100
NCHUNK = EPW // CHUNK
NB = 3
NR = 2 * NB
NP = 10240
ROWS_PER_TILE = NP // NS
ZCOPY = 80
ZROWS = 32


def _sc_partials(feature, src2, dst2):
    mesh = plsc.VectorSubcoreMesh(core_axis_name="c", subcore_axis_name="s")

    @functools.partial(
        pl.kernel,
        mesh=mesh,
        out_type=jax.ShapeDtypeStruct((NC, NP, D), jnp.float32),
        scratch_types=[
            pltpu.VMEM((NR, CHUNK), jnp.int32),
            pltpu.VMEM((NR, CHUNK), jnp.int32),
            pltpu.VMEM((NB, CHUNK, D), jnp.float32),
            pltpu.VMEM((ZROWS, D), jnp.float32),
            pltpu.VMEM_SHARED((NP, D), jnp.float32),
            [pltpu.SemaphoreType.DMA] * NR,
            [pltpu.SemaphoreType.DMA] * NB,
        ],
    )
    def k(feat_hbm, src_hbm, dst_hbm, out_hbm, sring, dring, rows, zbuf, acc,
          isems, gsems):
        cid = lax.axis_index("c")
        sid = lax.axis_index("s")
        wid = sid * NC + cid

        def load_idx(i, s):
            pltpu.async_copy(src_hbm.at[wid, i], sring.at[s], isems[s])
            pltpu.async_copy(dst_hbm.at[wid, i], dring.at[s], isems[s])

        def wait_idx(i, s):
            pltpu.make_async_copy(src_hbm.at[wid, i], sring.at[s], isems[s]).wait()
            pltpu.make_async_copy(dst_hbm.at[wid, i], dring.at[s], isems[s]).wait()

        def issue_g(b, s):
            pltpu.async_copy(feat_hbm.at[sring.at[s]], rows.at[b], gsems[b])

        def wait_g(b, s):
            pltpu.make_async_copy(feat_hbm.at[sring.at[s]], rows.at[b], gsems[b]).wait()

        def scatter(b, s):
            pltpu.sync_copy(rows.at[b], acc.at[dring.at[s]], add=True)

        for s in range(NR):
            load_idx(s, s)
        for b in range(NB):
            wait_idx(b, b)
            issue_g(b, b)

        def zrow(r, carry):
            def zcol(j, carry2):
                zbuf[r, pl.ds(j * 16, 16)] = jnp.zeros((16,), jnp.float32)
                return carry2
            return lax.fori_loop(0, D // 16, zcol, carry)
        lax.fori_loop(0, ZROWS, zrow, 0)
        for kk in range(ROWS_PER_TILE // ZROWS):
            pltpu.async_copy(zbuf, acc.at[pl.ds(sid * ROWS_PER_TILE + kk * ZROWS, ZROWS)],
                             isems[0])
        for kk in range(ROWS_PER_TILE // ZROWS):
            pltpu.make_async_copy(
                zbuf, acc.at[pl.ds(sid * ROWS_PER_TILE + kk * ZROWS, ZROWS)],
                isems[0]).wait()
        plsc.subcore_barrier()

        NGRP = (NCHUNK - NR) // NR

        def body(j, carry):
            a = NR * j
            for u in range(NR):
                b = u % NB
                wait_g(b, u)
                scatter(b, u)
                load_idx(a + u + NR, u)
                wait_idx(a + u + NB, (u + NB) % NR)
                issue_g(b, (u + NB) % NR)
            return carry
        lax.fori_loop(0, NGRP, body, 0)

        for i in range(NGRP * NR, NCHUNK):
            b = i % NB
            u = i % NR
            wait_g(b, u)
            scatter(b, u)
            if i + NR < NCHUNK:
                load_idx(i + NR, u)
            if i + NB < NCHUNK:
                wait_idx(i + NB, (i + NB) % NR)
                issue_g(b, (i + NB) % NR)
        plsc.subcore_barrier()

        for kk in range(ROWS_PER_TILE // ZCOPY):
            r0 = sid * ROWS_PER_TILE + kk * ZCOPY
            pltpu.async_copy(acc.at[pl.ds(r0, ZCOPY)],
                             out_hbm.at[cid, pl.ds(r0, ZCOPY)], gsems[0])
        for kk in range(ROWS_PER_TILE // ZCOPY):
            r0 = sid * ROWS_PER_TILE + kk * ZCOPY
            pltpu.make_async_copy(acc.at[pl.ds(r0, ZCOPY)],
                                  out_hbm.at[cid, pl.ds(r0, ZCOPY)], gsems[0]).wait()

    return k(feature, src2, dst2)


BLK = 2000


def _tc_linear_kernel(p_ref, w_ref, b_ref, out_ref):
    x = p_ref[0] + p_ref[1]
    y = lax.dot_general(
        x, w_ref[...], (((1,), (1,)), ((), ())),
        preferred_element_type=jnp.float32,
        precision=lax.Precision.HIGHEST,
    )
    out_ref[...] = y + b_ref[...]


def _tc_linear(partials, W, b):
    return pl.pallas_call(
        _tc_linear_kernel,
        grid=(N // BLK,),
        in_specs=[
            pl.BlockSpec((NC, BLK, D), lambda i: (0, i, 0)),
            pl.BlockSpec((D, D), lambda i: (0, 0)),
            pl.BlockSpec((1, D), lambda i: (0, 0)),
        ],
        out_specs=pl.BlockSpec((BLK, D), lambda i: (i, 0)),
        out_shape=jax.ShapeDtypeStruct((N, D), jnp.float32),
    )(partials, W, b.reshape(1, D))


@jax.jit
def kernel(feature, edge_index, W, b):
    src2 = edge_index[0].reshape(NW, NCHUNK, CHUNK)
    dst2 = edge_index[1].reshape(NW, NCHUNK, CHUNK)
    partials = _sc_partials(feature, src2, dst2)
    return _tc_linear(partials, W, b)


# speedup vs baseline: 1.1013x; 1.0145x over previous
import functools

import jax
import jax.numpy as jnp
from jax import lax
from jax.experimental import pallas as pl
from jax.experimental.pallas import tpu as pltpu
from jax.experimental.pallas import tpu_sc as plsc

N = 10000
E = 320000
D = 128

NC = 2
NS = 16
NW = NC * NS
EPW = E // NW
CHUNK = 100
NCHUNK = EPW // CHUNK
NB = 3
NR = 2 * NB
NP = 10240
ROWS_PER_TILE = NP // NS
ZCOPY = 80
ZROWS = 32


def _sc_partials(feature, src2, dst2):
    mesh = plsc.VectorSubcoreMesh(core_axis_name="c", subcore_axis_name="s")

    @functools.partial(
        pl.kernel,
        mesh=mesh,
        out_type=jax.ShapeDtypeStruct((NC, NP, D), jnp.float32),
        scratch_types=[
            pltpu.VMEM((NR, CHUNK), jnp.int32),
            pltpu.VMEM((NR, CHUNK), jnp.int32),
            pltpu.VMEM((NB, CHUNK, D), jnp.float32),
            pltpu.VMEM((ZROWS, D), jnp.float32),
            pltpu.VMEM_SHARED((NP, D), jnp.float32),
            [pltpu.SemaphoreType.DMA] * NR,
            [pltpu.SemaphoreType.DMA] * NB,
        ],
    )
    def k(feat_hbm, src_hbm, dst_hbm, out_hbm, sring, dring, rows, zbuf, acc,
          isems, gsems):
        cid = lax.axis_index("c")
        sid = lax.axis_index("s")
        wid = sid * NC + cid

        def load_idx(i, s):
            pltpu.async_copy(src_hbm.at[wid, i], sring.at[s], isems[s])
            pltpu.async_copy(dst_hbm.at[wid, i], dring.at[s], isems[s])

        def wait_idx(i, s):
            pltpu.make_async_copy(src_hbm.at[wid, i], sring.at[s], isems[s]).wait()
            pltpu.make_async_copy(dst_hbm.at[wid, i], dring.at[s], isems[s]).wait()

        def issue_g(b, s):
            pltpu.async_copy(feat_hbm.at[sring.at[s]], rows.at[b], gsems[b])

        def wait_g(b, s):
            pltpu.make_async_copy(feat_hbm.at[sring.at[s]], rows.at[b], gsems[b]).wait()

        def scatter(b, s):
            pltpu.sync_copy(rows.at[b], acc.at[dring.at[s]], add=True)

        for s in range(NR):
            load_idx(s, s)
        for b in range(NB):
            wait_idx(b, b)
            issue_g(b, b)

        def zrow(r, carry):
            def zcol(j, carry2):
                zbuf[r, pl.ds(j * 16, 16)] = jnp.zeros((16,), jnp.float32)
                return carry2
            return lax.fori_loop(0, D // 16, zcol, carry)
        lax.fori_loop(0, ZROWS, zrow, 0)
        for kk in range(ROWS_PER_TILE // ZROWS):
            pltpu.async_copy(zbuf, acc.at[pl.ds(sid * ROWS_PER_TILE + kk * ZROWS, ZROWS)],
                             isems[0])
        for kk in range(ROWS_PER_TILE // ZROWS):
            pltpu.make_async_copy(
                zbuf, acc.at[pl.ds(sid * ROWS_PER_TILE + kk * ZROWS, ZROWS)],
                isems[0]).wait()
        plsc.subcore_barrier()

        NGRP = (NCHUNK - NR) // NR

        def body(j, carry):
            a = NR * j
            for u in range(NR):
                b = u % NB
                wait_g(b, u)
                scatter(b, u)
                load_idx(a + u + NR, u)
                wait_idx(a + u + NB, (u + NB) % NR)
                issue_g(b, (u + NB) % NR)
            return carry
        lax.fori_loop(0, NGRP, body, 0)

        for i in range(NGRP * NR, NCHUNK):
            b = i % NB
            u = i % NR
            wait_g(b, u)
            scatter(b, u)
            if i + NR < NCHUNK:
                load_idx(i + NR, u)
            if i + NB < NCHUNK:
                wait_idx(i + NB, (i + NB) % NR)
                issue_g(b, (i + NB) % NR)
        plsc.subcore_barrier()

        for kk in range(ROWS_PER_TILE // ZCOPY):
            r0 = sid * ROWS_PER_TILE + kk * ZCOPY
            pltpu.async_copy(acc.at[pl.ds(r0, ZCOPY)],
                             out_hbm.at[cid, pl.ds(r0, ZCOPY)], gsems[0])
        for kk in range(ROWS_PER_TILE // ZCOPY):
            r0 = sid * ROWS_PER_TILE + kk * ZCOPY
            pltpu.make_async_copy(acc.at[pl.ds(r0, ZCOPY)],
                                  out_hbm.at[cid, pl.ds(r0, ZCOPY)], gsems[0]).wait()

    return k(feature, src2, dst2)


BLK = 2000


def _tc_linear_kernel(p_ref, w_ref, b_ref, out_ref):
    x = p_ref[0] + p_ref[1]
    y = lax.dot_general(
        x, w_ref[...], (((1,), (1,)), ((), ())),
        preferred_element_type=jnp.float32,
    )
    out_ref[...] = y + b_ref[...]


def _tc_linear(partials, W, b):
    return pl.pallas_call(
        _tc_linear_kernel,
        grid=(N // BLK,),
        in_specs=[
            pl.BlockSpec((NC, BLK, D), lambda i: (0, i, 0)),
            pl.BlockSpec((D, D), lambda i: (0, 0)),
            pl.BlockSpec((1, D), lambda i: (0, 0)),
        ],
        out_specs=pl.BlockSpec((BLK, D), lambda i: (i, 0)),
        out_shape=jax.ShapeDtypeStruct((N, D), jnp.float32),
    )(partials, W, b.reshape(1, D))


@jax.jit
def kernel(feature, edge_index, W, b):
    src2 = edge_index[0].reshape(NW, NCHUNK, CHUNK)
    dst2 = edge_index[1].reshape(NW, NCHUNK, CHUNK)
    partials = _sc_partials(feature, src2, dst2)
    return _tc_linear(partials, W, b)


# speedup vs baseline: 1.2008x; 1.0903x over previous
import functools

import jax
import jax.numpy as jnp
from jax import lax
from jax.experimental import pallas as pl
from jax.experimental.pallas import tpu as pltpu
from jax.experimental.pallas import tpu_sc as plsc

N = 10000
E = 320000
D = 128

NC = 2
NS = 16
NW = NC * NS
EPW = E // NW
CHUNK = 100
NCHUNK = EPW // CHUNK
NB = 3
NR = 2 * NB
NP = 10240
ROWS_PER_TILE = NP // NS
ZCOPY = 80
ZROWS = 32


def _sc_partials(feature, idx4):
    mesh = plsc.VectorSubcoreMesh(core_axis_name="c", subcore_axis_name="s")

    @functools.partial(
        pl.kernel,
        mesh=mesh,
        out_type=jax.ShapeDtypeStruct((NC, NP, D), jnp.float32),
        scratch_types=[
            pltpu.VMEM((NR, CHUNK), jnp.int32),
            pltpu.VMEM((NR, CHUNK), jnp.int32),
            pltpu.VMEM((NB, CHUNK, D), jnp.float32),
            pltpu.VMEM((ZROWS, D), jnp.float32),
            pltpu.VMEM_SHARED((NP, D), jnp.float32),
            [pltpu.SemaphoreType.DMA] * NR,
            [pltpu.SemaphoreType.DMA] * NB,
        ],
    )
    def k(feat_hbm, idx_hbm, out_hbm, sring, dring, rows, zbuf, acc,
          isems, gsems):
        cid = lax.axis_index("c")
        sid = lax.axis_index("s")
        wid = sid * NC + cid

        def load_idx(i, s):
            pltpu.async_copy(idx_hbm.at[0, wid, i], sring.at[s], isems[s])
            pltpu.async_copy(idx_hbm.at[1, wid, i], dring.at[s], isems[s])

        def wait_idx(i, s):
            pltpu.make_async_copy(idx_hbm.at[0, wid, i], sring.at[s], isems[s]).wait()
            pltpu.make_async_copy(idx_hbm.at[1, wid, i], dring.at[s], isems[s]).wait()

        def issue_g(b, s):
            pltpu.async_copy(feat_hbm.at[sring.at[s]], rows.at[b], gsems[b])

        def wait_g(b, s):
            pltpu.make_async_copy(feat_hbm.at[sring.at[s]], rows.at[b], gsems[b]).wait()

        def scatter(b, s):
            pltpu.sync_copy(rows.at[b], acc.at[dring.at[s]], add=True)

        for s in range(NR):
            load_idx(s, s)
        for b in range(NB):
            wait_idx(b, b)
            issue_g(b, b)

        def zrow(r, carry):
            def zcol(j, carry2):
                zbuf[r, pl.ds(j * 16, 16)] = jnp.zeros((16,), jnp.float32)
                return carry2
            return lax.fori_loop(0, D // 16, zcol, carry)
        lax.fori_loop(0, ZROWS, zrow, 0)
        for kk in range(ROWS_PER_TILE // ZROWS):
            pltpu.async_copy(zbuf, acc.at[pl.ds(sid * ROWS_PER_TILE + kk * ZROWS, ZROWS)],
                             isems[0])
        for kk in range(ROWS_PER_TILE // ZROWS):
            pltpu.make_async_copy(
                zbuf, acc.at[pl.ds(sid * ROWS_PER_TILE + kk * ZROWS, ZROWS)],
                isems[0]).wait()
        plsc.subcore_barrier()

        NGRP = (NCHUNK - NR) // NR

        def body(j, carry):
            a = NR * j
            for u in range(NR):
                b = u % NB
                wait_g(b, u)
                scatter(b, u)
                load_idx(a + u + NR, u)
                wait_idx(a + u + NB, (u + NB) % NR)
                issue_g(b, (u + NB) % NR)
            return carry
        lax.fori_loop(0, NGRP, body, 0)

        for i in range(NGRP * NR, NCHUNK):
            b = i % NB
            u = i % NR
            wait_g(b, u)
            scatter(b, u)
            if i + NR < NCHUNK:
                load_idx(i + NR, u)
            if i + NB < NCHUNK:
                wait_idx(i + NB, (i + NB) % NR)
                issue_g(b, (i + NB) % NR)
        plsc.subcore_barrier()

        for kk in range(ROWS_PER_TILE // ZCOPY):
            r0 = sid * ROWS_PER_TILE + kk * ZCOPY
            pltpu.async_copy(acc.at[pl.ds(r0, ZCOPY)],
                             out_hbm.at[cid, pl.ds(r0, ZCOPY)], gsems[0])
        for kk in range(ROWS_PER_TILE // ZCOPY):
            r0 = sid * ROWS_PER_TILE + kk * ZCOPY
            pltpu.make_async_copy(acc.at[pl.ds(r0, ZCOPY)],
                                  out_hbm.at[cid, pl.ds(r0, ZCOPY)], gsems[0]).wait()

    return k(feature, idx4)


BLK = 2000


def _tc_linear_kernel(p_ref, w_ref, b_ref, out_ref):
    x = p_ref[0] + p_ref[1]
    y = lax.dot_general(
        x, w_ref[...], (((1,), (1,)), ((), ())),
        preferred_element_type=jnp.float32,
    )
    out_ref[...] = y + b_ref[...]


def _tc_linear(partials, W, b):
    return pl.pallas_call(
        _tc_linear_kernel,
        grid=(N // BLK,),
        in_specs=[
            pl.BlockSpec((NC, BLK, D), lambda i: (0, i, 0)),
            pl.BlockSpec((D, D), lambda i: (0, 0)),
            pl.BlockSpec((1, D), lambda i: (0, 0)),
        ],
        out_specs=pl.BlockSpec((BLK, D), lambda i: (i, 0)),
        out_shape=jax.ShapeDtypeStruct((N, D), jnp.float32),
    )(partials, W, b.reshape(1, D))


@jax.jit
def kernel(feature, edge_index, W, b):
    idx4 = edge_index.reshape(2, NW, NCHUNK, CHUNK)
    partials = _sc_partials(feature, idx4)
    return _tc_linear(partials, W, b)


# speedup vs baseline: 1.2232x; 1.0187x over previous
import functools

import jax
import jax.numpy as jnp
from jax import lax
from jax.experimental import pallas as pl
from jax.experimental.pallas import tpu as pltpu
from jax.experimental.pallas import tpu_sc as plsc

N = 10000
E = 320000
D = 128

NC = 2
NS = 16
NW = NC * NS
EPW = E // NW
CHUNK = 100
NCHUNK = EPW // CHUNK
NB = 3
NR = 2 * NB
NP = 10240
ROWS_PER_TILE = NP // NS
ZCOPY = 80
ZROWS = 32


def _sc_partials(feature, idx4):
    mesh = plsc.VectorSubcoreMesh(core_axis_name="c", subcore_axis_name="s")

    @functools.partial(
        pl.kernel,
        mesh=mesh,
        out_type=jax.ShapeDtypeStruct((NC, NP, D), jnp.float32),
        scratch_types=[
            pltpu.VMEM((NR, CHUNK), jnp.int32),
            pltpu.VMEM((NR, CHUNK), jnp.int32),
            pltpu.VMEM((NB, CHUNK, D), jnp.float32),
            pltpu.VMEM((ZROWS, D), jnp.float32),
            pltpu.VMEM_SHARED((NP, D), jnp.float32),
            [pltpu.SemaphoreType.DMA] * NR,
            [pltpu.SemaphoreType.DMA] * NB,
        ],
    )
    def k(feat_hbm, idx_hbm, out_hbm, sring, dring, rows, zbuf, acc,
          isems, gsems):
        cid = lax.axis_index("c")
        sid = lax.axis_index("s")
        wid = sid * NC + cid

        def load_idx(i, s):
            pltpu.async_copy(idx_hbm.at[0, wid, i], sring.at[s], isems[s])
            pltpu.async_copy(idx_hbm.at[1, wid, i], dring.at[s], isems[s])

        def wait_idx(i, s):
            pltpu.make_async_copy(idx_hbm.at[0, wid, i], sring.at[s], isems[s]).wait()
            pltpu.make_async_copy(idx_hbm.at[1, wid, i], dring.at[s], isems[s]).wait()

        def issue_g(b, s):
            pltpu.async_copy(feat_hbm.at[sring.at[s]], rows.at[b], gsems[b])

        def wait_g(b, s):
            pltpu.make_async_copy(feat_hbm.at[sring.at[s]], rows.at[b], gsems[b]).wait()

        def scatter(b, s):
            pltpu.sync_copy(rows.at[b], acc.at[dring.at[s]], add=True)

        for s in range(NR):
            load_idx(s, s)
        for b in range(NB):
            wait_idx(b, b)
            issue_g(b, b)

        def zrow(r, carry):
            def zcol(j, carry2):
                zbuf[r, pl.ds(j * 16, 16)] = jnp.zeros((16,), jnp.float32)
                return carry2
            return lax.fori_loop(0, D // 16, zcol, carry)
        lax.fori_loop(0, ZROWS, zrow, 0)
        for kk in range(ROWS_PER_TILE // ZROWS):
            pltpu.async_copy(zbuf, acc.at[pl.ds(sid * ROWS_PER_TILE + kk * ZROWS, ZROWS)],
                             isems[0])
        for kk in range(ROWS_PER_TILE // ZROWS):
            pltpu.make_async_copy(
                zbuf, acc.at[pl.ds(sid * ROWS_PER_TILE + kk * ZROWS, ZROWS)],
                isems[0]).wait()
        plsc.subcore_barrier()

        NGRP = (NCHUNK - NR) // NR

        def body(j, carry):
            a = NR * j
            for u in range(NR):
                b = u % NB
                wait_g(b, u)
                scatter(b, u)
                load_idx(a + u + NR, u)
                wait_idx(a + u + NB, (u + NB) % NR)
                issue_g(b, (u + NB) % NR)
            return carry
        lax.fori_loop(0, NGRP, body, 0)

        for i in range(NGRP * NR, NCHUNK):
            b = i % NB
            u = i % NR
            wait_g(b, u)
            scatter(b, u)
            if i + NR < NCHUNK:
                load_idx(i + NR, u)
            if i + NB < NCHUNK:
                wait_idx(i + NB, (i + NB) % NR)
                issue_g(b, (i + NB) % NR)
        plsc.subcore_barrier()

        for kk in range(ROWS_PER_TILE // ZCOPY):
            r0 = sid * ROWS_PER_TILE + kk * ZCOPY
            pltpu.async_copy(acc.at[pl.ds(r0, ZCOPY)],
                             out_hbm.at[cid, pl.ds(r0, ZCOPY)], gsems[0])
        for kk in range(ROWS_PER_TILE // ZCOPY):
            r0 = sid * ROWS_PER_TILE + kk * ZCOPY
            pltpu.make_async_copy(acc.at[pl.ds(r0, ZCOPY)],
                                  out_hbm.at[cid, pl.ds(r0, ZCOPY)], gsems[0]).wait()

    return k(feature, idx4)


BLK = 5000


def _tc_linear_kernel(p_ref, w_ref, b_ref, out_ref):
    x = p_ref[0] + p_ref[1]
    y = lax.dot_general(
        x, w_ref[...], (((1,), (1,)), ((), ())),
        preferred_element_type=jnp.float32,
    )
    out_ref[...] = y + b_ref[...]


def _tc_linear(partials, W, b):
    return pl.pallas_call(
        _tc_linear_kernel,
        grid=(N // BLK,),
        in_specs=[
            pl.BlockSpec((NC, BLK, D), lambda i: (0, i, 0)),
            pl.BlockSpec((D, D), lambda i: (0, 0)),
            pl.BlockSpec((1, D), lambda i: (0, 0)),
        ],
        out_specs=pl.BlockSpec((BLK, D), lambda i: (i, 0)),
        out_shape=jax.ShapeDtypeStruct((N, D), jnp.float32),
    )(partials, W, b.reshape(1, D))


@jax.jit
def kernel(feature, edge_index, W, b):
    idx4 = edge_index.reshape(2, NW, NCHUNK, CHUNK)
    partials = _sc_partials(feature, idx4)
    return _tc_linear(partials, W, b)
